# baseline (speedup 1.0000x reference)
import functools

import jax
import jax.numpy as jnp
from jax import lax
from jax.experimental import pallas as pl
from jax.experimental.pallas import tpu as pltpu
from jax.experimental.pallas import tpu_sc as plsc

B = 16384
D = 64

_info = plsc.get_sparse_core_info()
NC, NS = _info.num_cores, _info.num_subcores
NW = NC * NS
BPW = B // NW


def _sc_gather(nodes, audio_tab, aid_tab, bid_tab, artist_tab, album_tab):
    mesh = plsc.VectorSubcoreMesh(core_axis_name="c", subcore_axis_name="s")

    @functools.partial(
        pl.kernel,
        mesh=mesh,
        compiler_params=pltpu.CompilerParams(use_tc_tiling_on_sc=False),
        out_type=(
            jax.ShapeDtypeStruct((B, D), jnp.bfloat16),
            jax.ShapeDtypeStruct((B, D), jnp.bfloat16),
            jax.ShapeDtypeStruct((B, D), jnp.bfloat16),
        ),
        scratch_types=[
            pltpu.VMEM((BPW,), jnp.int32),
            pltpu.VMEM((BPW,), jnp.int32),
            pltpu.VMEM((BPW,), jnp.int32),
            pltpu.VMEM((BPW, D), jnp.bfloat16),
            pltpu.VMEM((BPW, D), jnp.bfloat16),
            pltpu.VMEM((BPW, D), jnp.bfloat16),
            pltpu.SemaphoreType.DMA,
            pltpu.SemaphoreType.DMA,
            pltpu.SemaphoreType.DMA,
        ],
    )
    def k(nodes_hbm, audio_hbm, aid_hbm, bid_hbm, atab_hbm, btab_hbm,
          audio_out, art_out, alb_out,
          idx_v, aid_v, bid_v, audio_v, art_v, alb_v,
          sem_ids, sem_audio, sem_tab):
        wid = lax.axis_index("s") * NC + lax.axis_index("c")
        base = wid * BPW
        pltpu.sync_copy(nodes_hbm.at[pl.ds(base, BPW)], idx_v)
        c_audio = pltpu.async_copy(audio_hbm.at[idx_v], audio_v, sem_audio)
        c_aid = pltpu.async_copy(aid_hbm.at[idx_v], aid_v, sem_ids)
        c_bid = pltpu.async_copy(bid_hbm.at[idx_v], bid_v, sem_ids)
        c_aid.wait()
        c_bid.wait()
        c_art = pltpu.async_copy(atab_hbm.at[aid_v], art_v, sem_tab)
        c_alb = pltpu.async_copy(btab_hbm.at[bid_v], alb_v, sem_tab)
        c_audio.wait()
        pltpu.sync_copy(audio_v, audio_out.at[pl.ds(base, BPW)])
        c_art.wait()
        pltpu.sync_copy(art_v, art_out.at[pl.ds(base, BPW)])
        c_alb.wait()
        pltpu.sync_copy(alb_v, alb_out.at[pl.ds(base, BPW)])

    return k(nodes, audio_tab, aid_tab, bid_tab, artist_tab, album_tab)


BLK = 2048


def _tc_project(audio, art, alb, W, b2):
    def body(a_ref, r_ref, l_ref, w_ref, b_ref, o_ref):
        a = a_ref[...]
        m = r_ref[...] + l_ref[...]
        w = w_ref[...].astype(jnp.bfloat16)
        y = (jnp.dot(a, w[:D], preferred_element_type=jnp.float32)
             + jnp.dot(m, w[D:], preferred_element_type=jnp.float32)
             + b_ref[...])
        s = jnp.sum(y * y, axis=-1, keepdims=True)
        n = jnp.sqrt(s)
        o_ref[...] = y / jnp.maximum(n, 1e-12)

    return pl.pallas_call(
        body,
        grid=(B // BLK,),
        in_specs=[
            pl.BlockSpec((BLK, D), lambda i: (i, 0)),
            pl.BlockSpec((BLK, D), lambda i: (i, 0)),
            pl.BlockSpec((BLK, D), lambda i: (i, 0)),
            pl.BlockSpec((2 * D, D), lambda i: (0, 0)),
            pl.BlockSpec((1, D), lambda i: (0, 0)),
        ],
        out_specs=pl.BlockSpec((BLK, D), lambda i: (i, 0)),
        out_shape=jax.ShapeDtypeStruct((B, D), jnp.float32),
    )(audio, art, alb, W, b2)


def kernel(item_nodes, item_audio_emb, artist_ids, album_ids,
           artist_table, album_table, W, b):
    a_bf = item_audio_emb.astype(jnp.bfloat16)
    at_bf = artist_table.astype(jnp.bfloat16)
    al_bf = album_table.astype(jnp.bfloat16)
    audio, art, alb = _sc_gather(
        item_nodes.astype(jnp.int32), a_bf,
        artist_ids.astype(jnp.int32), album_ids.astype(jnp.int32),
        at_bf, al_bf)
    return _tc_project(audio, art, alb, W, b.reshape(1, D))


# speedup vs baseline: 1.2588x; 1.2588x over previous
import functools

import jax
import jax.numpy as jnp
from jax import lax
from jax.experimental import pallas as pl
from jax.experimental.pallas import tpu as pltpu
from jax.experimental.pallas import tpu_sc as plsc

B = 16384
D = 64

_info = plsc.get_sparse_core_info()
NC, NS = _info.num_cores, _info.num_subcores
NW = NC * NS
BPW = B // NW
NQ = BPW // 128
CH = 256
NCHK = BPW // CH
QC = CH // 128


def _sc_gather(nodes3, nodesh3, audio2, aid_tab, bid_tab, art2, alb2):
    mesh = plsc.VectorSubcoreMesh(core_axis_name="c", subcore_axis_name="s")

    @functools.partial(
        pl.kernel,
        mesh=mesh,
        out_type=(
            jax.ShapeDtypeStruct((B, 128), jnp.float32),
            jax.ShapeDtypeStruct((B, 128), jnp.float32),
            jax.ShapeDtypeStruct((B, 128), jnp.float32),
            jax.ShapeDtypeStruct((NW, NQ, 128), jnp.int32),
            jax.ShapeDtypeStruct((NW, NQ, 128), jnp.int32),
        ),
        scratch_types=[
            pltpu.VMEM((NQ, 128), jnp.int32),
            pltpu.VMEM((NQ, 128), jnp.int32),
            pltpu.VMEM((NQ, 128), jnp.int32),
            pltpu.VMEM((NQ, 128), jnp.int32),
            pltpu.VMEM((NQ, 128), jnp.int32),
            pltpu.VMEM((NQ, 128), jnp.int32),
            pltpu.VMEM((CH, 128), jnp.float32),
            pltpu.VMEM((CH, 128), jnp.float32),
            pltpu.VMEM((CH, 128), jnp.float32),
            pltpu.SemaphoreType.DMA,
            pltpu.SemaphoreType.DMA,
            pltpu.SemaphoreType.DMA,
        ],
    )
    def k(nodes_hbm, nodesh_hbm, audio_hbm, aid_hbm, bid_hbm, atab_hbm, btab_hbm,
          audio_out, art_out, alb_out, aid_out, bid_out,
          idx_v, idxh_v, aid_v, bid_v, aid2_v, bid2_v,
          audio_v, art_v, alb_v, sem_ids, sem_audio, sem_tab):
        wid = lax.axis_index("s") * NC + lax.axis_index("c")
        pltpu.sync_copy(nodes_hbm.at[wid], idx_v)
        pltpu.sync_copy(nodesh_hbm.at[wid], idxh_v)
        for h in range(NCHK):
            cbase = wid * BPW + h * CH
            audio_cps = []
            id_cps = []
            for qc in range(QC):
                q = h * QC + qc
                audio_cps.append(pltpu.async_copy(
                    audio_hbm.at[idxh_v.at[q]],
                    audio_v.at[pl.ds(qc * 128, 128)], sem_audio))
                id_cps.append(pltpu.async_copy(
                    aid_hbm.at[idx_v.at[q]], aid_v.at[q], sem_ids))
                id_cps.append(pltpu.async_copy(
                    bid_hbm.at[idx_v.at[q]], bid_v.at[q], sem_ids))
            for c in id_cps:
                c.wait()
            for qc in range(QC):
                q = h * QC + qc
                for j in range(8):
                    s = pl.ds(j * 16, 16)
                    aid2_v[q, s] = lax.shift_right_logical(aid_v[q, s], 1)
                    bid2_v[q, s] = lax.shift_right_logical(bid_v[q, s], 1)
            tab_cps = []
            for qc in range(QC):
                q = h * QC + qc
                tab_cps.append(pltpu.async_copy(
                    atab_hbm.at[aid2_v.at[q]],
                    art_v.at[pl.ds(qc * 128, 128)], sem_tab))
                tab_cps.append(pltpu.async_copy(
                    btab_hbm.at[bid2_v.at[q]],
                    alb_v.at[pl.ds(qc * 128, 128)], sem_tab))
            for c in audio_cps:
                c.wait()
            pltpu.sync_copy(audio_v, audio_out.at[pl.ds(cbase, CH)])
            for c in tab_cps:
                c.wait()
            pltpu.sync_copy(art_v, art_out.at[pl.ds(cbase, CH)])
            pltpu.sync_copy(alb_v, alb_out.at[pl.ds(cbase, CH)])
        pltpu.sync_copy(aid_v, aid_out.at[wid])
        pltpu.sync_copy(bid_v, bid_out.at[wid])

    return k(nodes3, nodesh3, audio2, aid_tab, bid_tab, art2, alb2)


BLK = 2048


def _tc_project(audioP, artP, albP, sa, ra, rb, W, b2):
    def body(a_ref, r_ref, l_ref, sa_ref, ra_ref, rb_ref, w_ref, b_ref, o_ref):
        a2 = a_ref[...]
        r2 = r_ref[...]
        l2 = l_ref[...]
        sa_ = sa_ref[...]
        ra_ = ra_ref[...]
        rb_ = rb_ref[...]
        a = a2[:, :D] + sa_ * (a2[:, D:] - a2[:, :D])
        r = r2[:, :D] + ra_ * (r2[:, D:] - r2[:, :D])
        l = l2[:, :D] + rb_ * (l2[:, D:] - l2[:, :D])
        m = r + l
        w = w_ref[...]
        y = (jnp.dot(a, w[:D], preferred_element_type=jnp.float32)
             + jnp.dot(m, w[D:], preferred_element_type=jnp.float32)
             + b_ref[...])
        s = jnp.sum(y * y, axis=-1, keepdims=True)
        n = jnp.sqrt(s)
        o_ref[...] = y / jnp.maximum(n, 1e-12)

    return pl.pallas_call(
        body,
        grid=(B // BLK,),
        in_specs=[
            pl.BlockSpec((BLK, 128), lambda i: (i, 0)),
            pl.BlockSpec((BLK, 128), lambda i: (i, 0)),
            pl.BlockSpec((BLK, 128), lambda i: (i, 0)),
            pl.BlockSpec((BLK, 1), lambda i: (i, 0)),
            pl.BlockSpec((BLK, 1), lambda i: (i, 0)),
            pl.BlockSpec((BLK, 1), lambda i: (i, 0)),
            pl.BlockSpec((2 * D, D), lambda i: (0, 0)),
            pl.BlockSpec((1, D), lambda i: (0, 0)),
        ],
        out_specs=pl.BlockSpec((BLK, D), lambda i: (i, 0)),
        out_shape=jax.ShapeDtypeStruct((B, D), jnp.float32),
    )(audioP, artP, albP, sa, ra, rb, W, b2)


def kernel(item_nodes, item_audio_emb, artist_ids, album_ids,
           artist_table, album_table, W, b):
    nodes = item_nodes.astype(jnp.int32)
    audio2 = item_audio_emb.reshape(item_audio_emb.shape[0] // 2, 128)
    art2 = artist_table.reshape(artist_table.shape[0] // 2, 128)
    alb2 = album_table.reshape(album_table.shape[0] // 2, 128)
    nodes3 = nodes.reshape(NW, NQ, 128)
    nodesh3 = (nodes >> 1).reshape(NW, NQ, 128)
    audioP, artP, albP, aidO, bidO = _sc_gather(
        nodes3, nodesh3, audio2,
        artist_ids.astype(jnp.int32), album_ids.astype(jnp.int32),
        art2, alb2)
    sa = (nodes & 1).astype(jnp.float32).reshape(B, 1)
    ra = (aidO & 1).astype(jnp.float32).reshape(B, 1)
    rb = (bidO & 1).astype(jnp.float32).reshape(B, 1)
    return _tc_project(audioP, artP, albP, sa, ra, rb, W, b.reshape(1, D))
